# baseline (device time: 74987 ns/iter reference)
import jax
import jax.numpy as jnp
from jax import lax
from jax.experimental import pallas as pl
from jax.experimental.pallas import tpu as pltpu

N_DEV = 16
P_PLANE = 4
N_Z = 4
N_H = 2


def kernel(t):
    m, n = t.shape
    q = m // P_PLANE
    c = q // N_Z
    w = n // N_H
    bf = jnp.bfloat16

    def body(t_ref, out_ref,
             tb, p1_recv, r_stage, z2_recv, y_stage, q_send,
             p1_ss, p1_rs, z2_ss, z2_rs, z4_ss, z4_rs, p5_ss, p5_rs):
        my = lax.axis_index("i")
        p = jnp.mod(my, P_PLANE)
        z = my // P_PLANE

        peers = [P_PLANE * z + jnp.mod(p + d, P_PLANE) for d in (1, 2, 3)]
        peers += [P_PLANE * jnp.mod(z + d, N_Z) + p for d in (1, 2, 3)]
        barrier_sem = pltpu.get_barrier_semaphore()
        for nbr in peers:
            pl.semaphore_signal(
                barrier_sem, inc=1,
                device_id=(nbr,), device_id_type=pl.DeviceIdType.MESH,
            )
        pl.semaphore_wait(barrier_sem, len(peers))

        cols = [slice(0, w), slice(w, n)]


        def p1_start(hi):
            quarters = []
            for qi in range(P_PLANE):
                v = t_ref[qi * q:(qi + 1) * q, cols[hi]].astype(bf)
                quarters.append(v)
                tb[hi, qi, :, :] = v
            rdmas = []
            for d in (1, 2, 3):
                tgt_p = jnp.mod(p + d, P_PLANE)
                rdma = pltpu.make_async_remote_copy(
                    src_ref=tb.at[hi].at[tgt_p],
                    dst_ref=p1_recv.at[hi].at[d - 1],
                    send_sem=p1_ss.at[hi, d - 1],
                    recv_sem=p1_rs.at[hi, d - 1],
                    device_id=(P_PLANE * z + tgt_p,),
                    device_id_type=pl.DeviceIdType.MESH,
                )
                rdma.start()
                rdmas.append(rdma)
            return rdmas, quarters

        def p2_start(hi, rdmas, quarters):
            for rdma in rdmas:
                rdma.wait()
            my_q = jnp.zeros((q, w), dtype=bf)
            for qi in range(P_PLANE):
                my_q = jnp.where(p == qi, quarters[qi], my_q)
            r32 = (my_q.astype(jnp.float32)
                   + p1_recv[hi, 0, :, :].astype(jnp.float32)
                   + p1_recv[hi, 1, :, :].astype(jnp.float32)
                   + p1_recv[hi, 2, :, :].astype(jnp.float32))
            r_val = r32.astype(bf)
            for s in range(N_Z):
                r_stage[hi, s, :, :] = r_val[s * c:(s + 1) * c, :]
            rdmas = []
            for d in (1, 2, 3):
                tgt_z = jnp.mod(z + d, N_Z)
                rdma = pltpu.make_async_remote_copy(
                    src_ref=r_stage.at[hi].at[tgt_z],
                    dst_ref=z2_recv.at[hi].at[d - 1],
                    send_sem=z2_ss.at[hi, d - 1],
                    recv_sem=z2_rs.at[hi, d - 1],
                    device_id=(P_PLANE * tgt_z + p,),
                    device_id_type=pl.DeviceIdType.MESH,
                )
                rdma.start()
                rdmas.append(rdma)
            own = jnp.zeros((c, w), dtype=bf)
            for s in range(N_Z):
                own = jnp.where(z == s, r_val[s * c:(s + 1) * c, :], own)
            return rdmas, own

        def p4_start(hi, rdmas, own):
            for rdma in rdmas:
                rdma.wait()
            s_val = (own.astype(jnp.float32)
                     + z2_recv[hi, 0, :, :].astype(jnp.float32)
                     + z2_recv[hi, 1, :, :].astype(jnp.float32)
                     + z2_recv[hi, 2, :, :].astype(jnp.float32))
            relu = jnp.maximum(s_val, 0.0)
            y = jnp.tanh(s_val) * s_val * s_val + relu * relu * relu
            y_bf = y.astype(bf)
            y_stage[hi, :, :] = y_bf
            rdmas = []
            for d in (1, 2, 3):
                tgt_z = jnp.mod(z + d, N_Z)
                rdma = pltpu.make_async_remote_copy(
                    src_ref=y_stage.at[hi],
                    dst_ref=q_send.at[hi, pl.ds(z * c, c), :],
                    send_sem=z4_ss.at[hi, d - 1],
                    recv_sem=z4_rs.at[hi, d - 1],
                    device_id=(P_PLANE * tgt_z + p,),
                    device_id_type=pl.DeviceIdType.MESH,
                )
                rdma.start()
                rdmas.append(rdma)
            q_send[hi, pl.ds(z * c, c), :] = y_bf
            return rdmas

        def p5_start(hi, rdmas):
            for rdma in rdmas:
                rdma.wait()
            rdmas = []
            for d in (1, 2, 3):
                tgt_p = jnp.mod(p + d, P_PLANE)
                rdma = pltpu.make_async_remote_copy(
                    src_ref=q_send.at[hi],
                    dst_ref=out_ref.at[pl.ds(p * q, q), cols[hi]],
                    send_sem=p5_ss.at[hi, d - 1],
                    recv_sem=p5_rs.at[hi, d - 1],
                    device_id=(P_PLANE * z + tgt_p,),
                    device_id_type=pl.DeviceIdType.MESH,
                )
                rdma.start()
                rdmas.append(rdma)
            out_ref[pl.ds(p * q, q), cols[hi]] = q_send[hi, :, :]
            return rdmas

        def p5_finish(rdmas):
            for rdma in rdmas:
                rdma.wait()

        p1a, qa = p1_start(0)
        p1b, qb = p1_start(1)
        p2a, own_a = p2_start(0, p1a, qa)
        p2b, own_b = p2_start(1, p1b, qb)
        p4a = p4_start(0, p2a, own_a)
        p4b = p4_start(1, p2b, own_b)
        p5a = p5_start(0, p4a)
        p5b = p5_start(1, p4b)
        p5_finish(p5a)
        p5_finish(p5b)

    return pl.pallas_call(
        body,
        out_shape=jax.ShapeDtypeStruct((m, n), bf),
        in_specs=[pl.BlockSpec(memory_space=pltpu.VMEM)],
        out_specs=pl.BlockSpec(memory_space=pltpu.VMEM),
        scratch_shapes=[
            pltpu.VMEM((N_H, P_PLANE, q, w), bf),
            pltpu.VMEM((N_H, 3, q, w), bf),
            pltpu.VMEM((N_H, N_Z, c, w), bf),
            pltpu.VMEM((N_H, 3, c, w), bf),
            pltpu.VMEM((N_H, c, w), bf),
            pltpu.VMEM((N_H, q, w), bf),
            pltpu.SemaphoreType.DMA((N_H, 3)),
            pltpu.SemaphoreType.DMA((N_H, 3)),
            pltpu.SemaphoreType.DMA((N_H, 3)),
            pltpu.SemaphoreType.DMA((N_H, 3)),
            pltpu.SemaphoreType.DMA((N_H, 3)),
            pltpu.SemaphoreType.DMA((N_H, 3)),
            pltpu.SemaphoreType.DMA((N_H, 3)),
            pltpu.SemaphoreType.DMA((N_H, 3)),
        ],
        compiler_params=pltpu.CompilerParams(collective_id=0),
    )(t)


# device time: 69353 ns/iter; 1.0812x vs baseline; 1.0812x over previous
import jax
import jax.numpy as jnp
from jax import lax
from jax.experimental import pallas as pl
from jax.experimental.pallas import tpu as pltpu

N_DEV = 16
P_PLANE = 4
N_Z = 4
N_H = 4


def kernel(t):
    m, n = t.shape
    q = m // P_PLANE
    c = q // N_Z
    w = n // N_H
    bf = jnp.bfloat16

    def body(t_ref, out_ref,
             tv, tb, p1_recv, r_stage, z2_recv, y_stage, q_send,
             load_sem, p1_ss, p1_rs, z2_ss, z2_rs, z4_ss, z4_rs,
             p5_ss, p5_rs):
        my = lax.axis_index("i")
        p = jnp.mod(my, P_PLANE)
        z = my // P_PLANE

        peers = [P_PLANE * z + jnp.mod(p + d, P_PLANE) for d in (1, 2, 3)]
        peers += [P_PLANE * jnp.mod(z + d, N_Z) + p for d in (1, 2, 3)]
        barrier_sem = pltpu.get_barrier_semaphore()
        for nbr in peers:
            pl.semaphore_signal(
                barrier_sem, inc=1,
                device_id=(nbr,), device_id_type=pl.DeviceIdType.MESH,
            )
        pl.semaphore_wait(barrier_sem, len(peers))

        cols = [slice(hi * w, (hi + 1) * w) for hi in range(N_H)]

        loads = []
        for hi in range(N_H):
            cp = pltpu.make_async_copy(
                t_ref.at[:, cols[hi]], tv.at[hi], load_sem.at[hi])
            cp.start()
            loads.append(cp)


        def p1_start(hi):
            loads[hi].wait()
            quarters = []
            for qi in range(P_PLANE):
                v = tv[hi, qi * q:(qi + 1) * q, :].astype(bf)
                quarters.append(v)
                tb[hi, qi, :, :] = v
            rdmas = []
            for d in (1, 2, 3):
                tgt_p = jnp.mod(p + d, P_PLANE)
                rdma = pltpu.make_async_remote_copy(
                    src_ref=tb.at[hi].at[tgt_p],
                    dst_ref=p1_recv.at[hi].at[d - 1],
                    send_sem=p1_ss.at[hi, d - 1],
                    recv_sem=p1_rs.at[hi, d - 1],
                    device_id=(P_PLANE * z + tgt_p,),
                    device_id_type=pl.DeviceIdType.MESH,
                )
                rdma.start()
                rdmas.append(rdma)
            return rdmas, quarters

        def p2_start(hi, rdmas, quarters):
            for rdma in rdmas:
                rdma.wait()
            my_q = jnp.zeros((q, w), dtype=bf)
            for qi in range(P_PLANE):
                my_q = jnp.where(p == qi, quarters[qi], my_q)
            r32 = (my_q.astype(jnp.float32)
                   + p1_recv[hi, 0, :, :].astype(jnp.float32)
                   + p1_recv[hi, 1, :, :].astype(jnp.float32)
                   + p1_recv[hi, 2, :, :].astype(jnp.float32))
            r_val = r32.astype(bf)
            for s in range(N_Z):
                r_stage[hi, s, :, :] = r_val[s * c:(s + 1) * c, :]
            rdmas = []
            for d in (1, 2, 3):
                tgt_z = jnp.mod(z + d, N_Z)
                rdma = pltpu.make_async_remote_copy(
                    src_ref=r_stage.at[hi].at[tgt_z],
                    dst_ref=z2_recv.at[hi].at[d - 1],
                    send_sem=z2_ss.at[hi, d - 1],
                    recv_sem=z2_rs.at[hi, d - 1],
                    device_id=(P_PLANE * tgt_z + p,),
                    device_id_type=pl.DeviceIdType.MESH,
                )
                rdma.start()
                rdmas.append(rdma)
            own = jnp.zeros((c, w), dtype=bf)
            for s in range(N_Z):
                own = jnp.where(z == s, r_val[s * c:(s + 1) * c, :], own)
            return rdmas, own

        def p4_start(hi, rdmas, own):
            for rdma in rdmas:
                rdma.wait()
            s_val = (own.astype(jnp.float32)
                     + z2_recv[hi, 0, :, :].astype(jnp.float32)
                     + z2_recv[hi, 1, :, :].astype(jnp.float32)
                     + z2_recv[hi, 2, :, :].astype(jnp.float32))
            relu = jnp.maximum(s_val, 0.0)
            y = jnp.tanh(s_val) * s_val * s_val + relu * relu * relu
            y_bf = y.astype(bf)
            y_stage[hi, :, :] = y_bf
            rdmas = []
            for d in (1, 2, 3):
                tgt_z = jnp.mod(z + d, N_Z)
                rdma = pltpu.make_async_remote_copy(
                    src_ref=y_stage.at[hi],
                    dst_ref=q_send.at[hi, pl.ds(z * c, c), :],
                    send_sem=z4_ss.at[hi, d - 1],
                    recv_sem=z4_rs.at[hi, d - 1],
                    device_id=(P_PLANE * tgt_z + p,),
                    device_id_type=pl.DeviceIdType.MESH,
                )
                rdma.start()
                rdmas.append(rdma)
            q_send[hi, pl.ds(z * c, c), :] = y_bf
            return rdmas

        def p5_start(hi, rdmas):
            for rdma in rdmas:
                rdma.wait()
            rdmas = []
            for d in (1, 2, 3):
                tgt_p = jnp.mod(p + d, P_PLANE)
                rdma = pltpu.make_async_remote_copy(
                    src_ref=q_send.at[hi],
                    dst_ref=out_ref.at[pl.ds(p * q, q), cols[hi]],
                    send_sem=p5_ss.at[hi, d - 1],
                    recv_sem=p5_rs.at[hi, d - 1],
                    device_id=(P_PLANE * z + tgt_p,),
                    device_id_type=pl.DeviceIdType.MESH,
                )
                rdma.start()
                rdmas.append(rdma)
            out_ref[pl.ds(p * q, q), cols[hi]] = q_send[hi, :, :]
            return rdmas

        def p5_finish(rdmas):
            for rdma in rdmas:
                rdma.wait()

        st = {}
        for hi in range(N_H):
            st[hi] = p1_start(hi)
        for hi in range(N_H):
            st[hi] = p2_start(hi, *st[hi])
        for hi in range(N_H):
            st[hi] = p4_start(hi, *st[hi])
        for hi in range(N_H):
            st[hi] = p5_start(hi, st[hi])
        for hi in range(N_H):
            p5_finish(st[hi])

    return pl.pallas_call(
        body,
        out_shape=jax.ShapeDtypeStruct((m, n), bf),
        in_specs=[pl.BlockSpec(memory_space=pltpu.MemorySpace.HBM)],
        out_specs=pl.BlockSpec(memory_space=pltpu.VMEM),
        scratch_shapes=[
            pltpu.VMEM((N_H, m, w), jnp.float32),
            pltpu.VMEM((N_H, P_PLANE, q, w), bf),
            pltpu.VMEM((N_H, 3, q, w), bf),
            pltpu.VMEM((N_H, N_Z, c, w), bf),
            pltpu.VMEM((N_H, 3, c, w), bf),
            pltpu.VMEM((N_H, c, w), bf),
            pltpu.VMEM((N_H, q, w), bf),
            pltpu.SemaphoreType.DMA((N_H,)),
            pltpu.SemaphoreType.DMA((N_H, 3)),
            pltpu.SemaphoreType.DMA((N_H, 3)),
            pltpu.SemaphoreType.DMA((N_H, 3)),
            pltpu.SemaphoreType.DMA((N_H, 3)),
            pltpu.SemaphoreType.DMA((N_H, 3)),
            pltpu.SemaphoreType.DMA((N_H, 3)),
            pltpu.SemaphoreType.DMA((N_H, 3)),
            pltpu.SemaphoreType.DMA((N_H, 3)),
        ],
        compiler_params=pltpu.CompilerParams(collective_id=0),
    )(t)


# device time: 66903 ns/iter; 1.1208x vs baseline; 1.0366x over previous
import jax
import jax.numpy as jnp
from jax import lax
from jax.experimental import pallas as pl
from jax.experimental.pallas import tpu as pltpu

N_DEV = 16
P_PLANE = 4
N_Z = 4
N_H = 8


def kernel(t):
    m, n = t.shape
    q = m // P_PLANE
    c = q // N_Z
    w = n // N_H
    bf = jnp.bfloat16

    def body(t_ref, out_ref,
             tv, tb, p1_recv, r_stage, z2_recv, y_stage, q_send,
             load_sem, p1_ss, p1_rs, z2_ss, z2_rs, z4_ss, z4_rs,
             p5_ss, p5_rs):
        my = lax.axis_index("i")
        p = jnp.mod(my, P_PLANE)
        z = my // P_PLANE

        peers = [P_PLANE * z + jnp.mod(p + d, P_PLANE) for d in (1, 2, 3)]
        peers += [P_PLANE * jnp.mod(z + d, N_Z) + p for d in (1, 2, 3)]
        barrier_sem = pltpu.get_barrier_semaphore()
        for nbr in peers:
            pl.semaphore_signal(
                barrier_sem, inc=1,
                device_id=(nbr,), device_id_type=pl.DeviceIdType.MESH,
            )
        pl.semaphore_wait(barrier_sem, len(peers))

        cols = [slice(hi * w, (hi + 1) * w) for hi in range(N_H)]

        loads = []
        for hi in range(N_H):
            cp = pltpu.make_async_copy(
                t_ref.at[:, cols[hi]], tv.at[hi], load_sem.at[hi])
            cp.start()
            loads.append(cp)


        def p1_start(hi):
            loads[hi].wait()
            quarters = []
            for qi in range(P_PLANE):
                v = tv[hi, qi * q:(qi + 1) * q, :].astype(bf)
                quarters.append(v)
                tb[hi, qi, :, :] = v
            rdmas = []
            for d in (1, 2, 3):
                tgt_p = jnp.mod(p + d, P_PLANE)
                rdma = pltpu.make_async_remote_copy(
                    src_ref=tb.at[hi].at[tgt_p],
                    dst_ref=p1_recv.at[hi].at[d - 1],
                    send_sem=p1_ss.at[hi, d - 1],
                    recv_sem=p1_rs.at[hi, d - 1],
                    device_id=(P_PLANE * z + tgt_p,),
                    device_id_type=pl.DeviceIdType.MESH,
                )
                rdma.start()
                rdmas.append(rdma)
            return rdmas, quarters

        def p2_start(hi, rdmas, quarters):
            for rdma in rdmas:
                rdma.wait()
            my_q = jnp.zeros((q, w), dtype=bf)
            for qi in range(P_PLANE):
                my_q = jnp.where(p == qi, quarters[qi], my_q)
            r32 = (my_q.astype(jnp.float32)
                   + p1_recv[hi, 0, :, :].astype(jnp.float32)
                   + p1_recv[hi, 1, :, :].astype(jnp.float32)
                   + p1_recv[hi, 2, :, :].astype(jnp.float32))
            r_val = r32.astype(bf)
            for s in range(N_Z):
                r_stage[hi, s, :, :] = r_val[s * c:(s + 1) * c, :]
            rdmas = []
            for d in (1, 2, 3):
                tgt_z = jnp.mod(z + d, N_Z)
                rdma = pltpu.make_async_remote_copy(
                    src_ref=r_stage.at[hi].at[tgt_z],
                    dst_ref=z2_recv.at[hi].at[d - 1],
                    send_sem=z2_ss.at[hi, d - 1],
                    recv_sem=z2_rs.at[hi, d - 1],
                    device_id=(P_PLANE * tgt_z + p,),
                    device_id_type=pl.DeviceIdType.MESH,
                )
                rdma.start()
                rdmas.append(rdma)
            own = jnp.zeros((c, w), dtype=bf)
            for s in range(N_Z):
                own = jnp.where(z == s, r_val[s * c:(s + 1) * c, :], own)
            return rdmas, own

        def p4_start(hi, rdmas, own):
            for rdma in rdmas:
                rdma.wait()
            s_val = (own.astype(jnp.float32)
                     + z2_recv[hi, 0, :, :].astype(jnp.float32)
                     + z2_recv[hi, 1, :, :].astype(jnp.float32)
                     + z2_recv[hi, 2, :, :].astype(jnp.float32))
            relu = jnp.maximum(s_val, 0.0)
            y = jnp.tanh(s_val) * s_val * s_val + relu * relu * relu
            y_bf = y.astype(bf)
            y_stage[hi, :, :] = y_bf
            rdmas = []
            for d in (1, 2, 3):
                tgt_z = jnp.mod(z + d, N_Z)
                rdma = pltpu.make_async_remote_copy(
                    src_ref=y_stage.at[hi],
                    dst_ref=q_send.at[hi, pl.ds(z * c, c), :],
                    send_sem=z4_ss.at[hi, d - 1],
                    recv_sem=z4_rs.at[hi, d - 1],
                    device_id=(P_PLANE * tgt_z + p,),
                    device_id_type=pl.DeviceIdType.MESH,
                )
                rdma.start()
                rdmas.append(rdma)
            q_send[hi, pl.ds(z * c, c), :] = y_bf
            return rdmas

        def p5_start(hi, rdmas):
            for rdma in rdmas:
                rdma.wait()
            rdmas = []
            for d in (1, 2, 3):
                tgt_p = jnp.mod(p + d, P_PLANE)
                rdma = pltpu.make_async_remote_copy(
                    src_ref=q_send.at[hi],
                    dst_ref=out_ref.at[pl.ds(p * q, q), cols[hi]],
                    send_sem=p5_ss.at[hi, d - 1],
                    recv_sem=p5_rs.at[hi, d - 1],
                    device_id=(P_PLANE * z + tgt_p,),
                    device_id_type=pl.DeviceIdType.MESH,
                )
                rdma.start()
                rdmas.append(rdma)
            out_ref[pl.ds(p * q, q), cols[hi]] = q_send[hi, :, :]
            return rdmas

        def p5_finish(rdmas):
            for rdma in rdmas:
                rdma.wait()

        st = {}
        for hi in range(N_H):
            st[hi] = p1_start(hi)
        for hi in range(N_H):
            st[hi] = p2_start(hi, *st[hi])
        for hi in range(N_H):
            st[hi] = p4_start(hi, *st[hi])
        for hi in range(N_H):
            st[hi] = p5_start(hi, st[hi])
        for hi in range(N_H):
            p5_finish(st[hi])

    return pl.pallas_call(
        body,
        out_shape=jax.ShapeDtypeStruct((m, n), bf),
        in_specs=[pl.BlockSpec(memory_space=pltpu.MemorySpace.HBM)],
        out_specs=pl.BlockSpec(memory_space=pltpu.VMEM),
        scratch_shapes=[
            pltpu.VMEM((N_H, m, w), jnp.float32),
            pltpu.VMEM((N_H, P_PLANE, q, w), bf),
            pltpu.VMEM((N_H, 3, q, w), bf),
            pltpu.VMEM((N_H, N_Z, c, w), bf),
            pltpu.VMEM((N_H, 3, c, w), bf),
            pltpu.VMEM((N_H, c, w), bf),
            pltpu.VMEM((N_H, q, w), bf),
            pltpu.SemaphoreType.DMA((N_H,)),
            pltpu.SemaphoreType.DMA((N_H, 3)),
            pltpu.SemaphoreType.DMA((N_H, 3)),
            pltpu.SemaphoreType.DMA((N_H, 3)),
            pltpu.SemaphoreType.DMA((N_H, 3)),
            pltpu.SemaphoreType.DMA((N_H, 3)),
            pltpu.SemaphoreType.DMA((N_H, 3)),
            pltpu.SemaphoreType.DMA((N_H, 3)),
            pltpu.SemaphoreType.DMA((N_H, 3)),
        ],
        compiler_params=pltpu.CompilerParams(collective_id=0),
    )(t)
